# baseline (speedup 1.0000x reference)
import functools

import jax
import jax.numpy as jnp
from jax import lax
from jax.experimental import pallas as pl
from jax.experimental.pallas import tpu as pltpu
from jax.experimental.pallas import tpu_sc as plsc

_S = 64
_K = 8
_RADIUS = 2.0
_R = 2.0 * _RADIUS / float(_S)
_R2 = _R * _R
_B = 2
_N = 8192
_F = 64
_P = _S * _S
_NB = 16
_TP = _P // _NB
_CAPL = 192
_LSTRIDE = 200
_CAP = 16 * _LSTRIDE
_GCH = 128
_MARGIN = 0.15625


def _sc_bin_body(xs_hbm, ys_hbm, zs_hbm, featsp_hbm,
                 bx_hbm, by_hbm, bz_hbm, bi_hbm, bandfeat_hbm,
                 xs_v, ys_v, zs_v, bx_v, by_v, bz_v, bi_v, gch_v, fbuf_v, sem):
    b = lax.axis_index("c")
    t = lax.axis_index("s")
    pltpu.sync_copy(xs_hbm.at[b, 0], xs_v)
    pltpu.sync_copy(ys_hbm.at[b, 0], ys_v)
    pltpu.sync_copy(zs_hbm.at[b, 0], zs_v)
    tf = t.astype(jnp.float32)
    hi = (1.0 - (8.0 * tf + 1.0) * 0.015625) + _MARGIN
    lo = (1.0 - (8.0 * tf + 7.0) * 0.015625) - _MARGIN
    base = b * _N

    zeros16 = jnp.zeros((16,), jnp.float32)
    negone16 = jnp.full((16,), -1.0, jnp.float32)
    base16 = jnp.zeros((16,), jnp.int32) + base

    def prefill(j, carry):
        sl = pl.ds(j * 16, 16)
        bx_v[sl] = zeros16
        by_v[sl] = zeros16
        bz_v[sl] = negone16
        bi_v[sl] = base16
        return carry
    lax.fori_loop(0, _CAP // 16, prefill, jnp.int32(0))

    iota16 = lax.iota(jnp.int32, 16)

    lane_base = iota16 * _LSTRIDE

    def scan(i, cntv):
        sl = pl.ds(i * 16, 16)
        y = ys_v[sl]
        m = (y >= lo) & (y <= hi)
        pos = lane_base + jnp.where(m, jnp.minimum(cntv, _CAPL - 1), _CAPL)
        plsc.store_scatter(bx_v, [pos], xs_v[sl])
        plsc.store_scatter(by_v, [pos], y)
        plsc.store_scatter(bz_v, [pos], zs_v[sl])
        plsc.store_scatter(bi_v, [pos], base16 + (i * 16 + iota16))
        return cntv + m.astype(jnp.int32)

    lax.fori_loop(0, _N // 16, scan, jnp.zeros((16,), jnp.int32))

    lane_base = iota16 * _LSTRIDE


    pltpu.sync_copy(bx_v.at[pl.ds(0, _CAP)], bx_hbm.at[b, t, 0])
    pltpu.sync_copy(by_v.at[pl.ds(0, _CAP)], by_hbm.at[b, t, 0])
    pltpu.sync_copy(bz_v.at[pl.ds(0, _CAP)], bz_hbm.at[b, t, 0])
    pltpu.sync_copy(bi_v.at[pl.ds(0, _CAP)], bi_hbm.at[b, t, 0])
    for ch in range(_CAP // _GCH):
        idx_sl = bi_v.at[pl.ds(ch * _GCH, _GCH)]
        pltpu.async_copy(featsp_hbm.at[idx_sl], fbuf_v, sem).wait()
        pltpu.sync_copy(fbuf_v, bandfeat_hbm.at[b, t, pl.ds(ch * _GCH, _GCH)])


def _sc_bin(xs, ys, zs, featsp):
    mesh = plsc.VectorSubcoreMesh(core_axis_name="c", subcore_axis_name="s")
    out_type = (
        jax.ShapeDtypeStruct((_B, _NB, 1, _CAP), jnp.float32),
        jax.ShapeDtypeStruct((_B, _NB, 1, _CAP), jnp.float32),
        jax.ShapeDtypeStruct((_B, _NB, 1, _CAP), jnp.float32),
        jax.ShapeDtypeStruct((_B, _NB, 1, _CAP), jnp.int32),
        jax.ShapeDtypeStruct((_B, _NB, _CAP, 2 * _F), jnp.float32),
    )
    scratch_types = [
        pltpu.VMEM((_N,), jnp.float32),
        pltpu.VMEM((_N,), jnp.float32),
        pltpu.VMEM((_N,), jnp.float32),
        pltpu.VMEM((_CAP,), jnp.float32),
        pltpu.VMEM((_CAP,), jnp.float32),
        pltpu.VMEM((_CAP,), jnp.float32),
        pltpu.VMEM((_CAP,), jnp.int32),
        pltpu.VMEM((_GCH,), jnp.int32),
        pltpu.VMEM((_GCH, 2 * _F), jnp.float32),
        pltpu.SemaphoreType.DMA,
    ]
    fn = functools.partial(
        pl.kernel, mesh=mesh, out_type=out_type, scratch_types=scratch_types,
        compiler_params=pltpu.CompilerParams(needs_layout_passes=False),
    )(_sc_bin_body)
    return fn(xs, ys, zs, featsp)


def _raster_body(bx_ref, by_ref, bz_ref, bi_ref, bandfeat_ref, fout_ref,
                 zw_ref, vray_ref, idx_ref, zbuf_ref, dist_ref, w_ref):
    t = pl.program_id(1)

    pix_lin = lax.broadcasted_iota(jnp.int32, (_TP, 1), 0)
    row = t * 4 + pix_lin // _S
    col = pix_lin % _S
    pixx = 1.0 - (2.0 * col.astype(jnp.float32) + 1.0) / float(_S)
    pixy = 1.0 - (2.0 * row.astype(jnp.float32) + 1.0) / float(_S)

    ptx = bx_ref[0, 0]
    pty = by_ref[0, 0]
    ptz = bz_ref[0, 0]
    gidx = bi_ref[0, 0]

    a = pixx * pixx + pixy * pixy
    bb = ptx * ptx + pty * pty
    bf = lambda v: v.astype(jnp.bfloat16).astype(jnp.float32)
    c = bf(pixx) * bf(ptx) + bf(pixy) * bf(pty)
    d2 = (a + bb) - 2.0 * c

    valid = (d2 <= _R2) & (ptz > 0.0)
    zbig = jnp.where(valid, jnp.broadcast_to(ptz, d2.shape), jnp.inf)
    big = jnp.int32(1 << 30)

    wmat = jnp.zeros((_TP, _CAP), jnp.float32)
    idx_cols = []
    zbuf_cols = []
    dist_cols = []
    w_cols = []
    found0 = None
    for _ in range(_K):
        zmin = jnp.min(zbig, axis=1, keepdims=True)
        found = zmin < jnp.inf
        eq = zbig == zmin
        amin = jnp.min(jnp.where(eq, gidx, big), axis=1,
                       keepdims=True)
        onehot = eq & (gidx == amin)
        d2sel = jnp.max(jnp.where(onehot, d2, -1.0), axis=1,
                        keepdims=True)
        zbig = jnp.where(onehot, jnp.inf, zbig)
        wsel = jnp.clip(jnp.exp(-jnp.maximum(d2sel / _R2, 0.0)), 0.0, 0.99)
        wmat = jnp.where(onehot & found, wsel, wmat)
        idx_cols.append(jnp.where(found, amin, -1))
        zbuf_cols.append(jnp.where(found, zmin, -1.0))
        dist_cols.append(jnp.where(found, d2sel, -1.0))
        w_cols.append(jnp.where(found, wsel, 0.99))
        if found0 is None:
            found0 = found

    zbuf_tk = jnp.concatenate(zbuf_cols, axis=1)
    dist_tk = jnp.concatenate(dist_cols, axis=1)
    idx_tk = jnp.concatenate(idx_cols, axis=1)
    w_tk = jnp.concatenate(w_cols, axis=1)

    fout = lax.dot_general(bandfeat_ref[0, 0], wmat,
                           dimension_numbers=(((0,), (1,)), ((), ())),
                           precision=lax.Precision.HIGHEST,
                           preferred_element_type=jnp.float32)
    fout_ref[0] = fout[0:_F, :]

    wn = jnp.where(idx_tk >= 0, w_tk, 0.0)
    denom = jnp.maximum(jnp.sum(wn, axis=1, keepdims=True), 1e-9)
    wn = wn / denom
    zw = jnp.sum(zbuf_tk * wn, axis=1, keepdims=True)

    zw_ref[0] = zw.reshape(1, _TP)
    vray_ref[0] = found0.astype(jnp.float32).reshape(1, _TP)
    idx_ref[0] = idx_tk.T
    zbuf_ref[0] = zbuf_tk
    dist_ref[0] = dist_tk
    w_ref[0] = w_tk.T


@jax.jit
def kernel(points, features):
    xs = points[:, :, 0].reshape(_B, 1, _N)
    ys = points[:, :, 1].reshape(_B, 1, _N)
    zs = points[:, :, 2].reshape(_B, 1, _N)
    featsp = jnp.pad(features.reshape(_B * _N, _F), ((0, 0), (0, _F)))

    bx, by, bz, bi, bandfeat = _sc_bin(xs, ys, zs, featsp)

    grid = (_B, _NB)
    out_shapes = (
        jax.ShapeDtypeStruct((_B, _F, _P), jnp.float32),
        jax.ShapeDtypeStruct((_B, 1, _P), jnp.float32),
        jax.ShapeDtypeStruct((_B, 1, _P), jnp.float32),
        jax.ShapeDtypeStruct((_B, _K, _P), jnp.int32),
        jax.ShapeDtypeStruct((_B, _P, _K), jnp.float32),
        jax.ShapeDtypeStruct((_B, _P, _K), jnp.float32),
        jax.ShapeDtypeStruct((_B, _K, _P), jnp.float32),
    )
    in_specs = [
        pl.BlockSpec((1, 1, 1, _CAP), lambda b, t: (b, t, 0, 0)),
        pl.BlockSpec((1, 1, 1, _CAP), lambda b, t: (b, t, 0, 0)),
        pl.BlockSpec((1, 1, 1, _CAP), lambda b, t: (b, t, 0, 0)),
        pl.BlockSpec((1, 1, 1, _CAP), lambda b, t: (b, t, 0, 0)),
        pl.BlockSpec((1, 1, _CAP, 2 * _F), lambda b, t: (b, t, 0, 0)),
    ]
    out_specs = (
        pl.BlockSpec((1, _F, _TP), lambda b, t: (b, 0, t)),
        pl.BlockSpec((1, 1, _TP), lambda b, t: (b, 0, t)),
        pl.BlockSpec((1, 1, _TP), lambda b, t: (b, 0, t)),
        pl.BlockSpec((1, _K, _TP), lambda b, t: (b, 0, t)),
        pl.BlockSpec((1, _TP, _K), lambda b, t: (b, t, 0)),
        pl.BlockSpec((1, _TP, _K), lambda b, t: (b, t, 0)),
        pl.BlockSpec((1, _K, _TP), lambda b, t: (b, 0, t)),
    )
    fout, zw, vray, idx, zbuf, dist, w = pl.pallas_call(
        _raster_body,
        grid=grid,
        in_specs=in_specs,
        out_specs=out_specs,
        out_shape=out_shapes,
    )(bx, by, bz, bi, bandfeat)

    feats_out = fout.reshape(_B, _F, _S, _S)
    z_weighted = zw.reshape(_B, 1, _S, _S)
    valid_ray = vray.reshape(_B, _S, _S)
    idx_o = idx.reshape(_B, _K, _S, _S)
    zbuf_o = zbuf.reshape(_B, _S, _S, _K)
    dist_o = dist.reshape(_B, _S, _S, _K)
    w_o = w.reshape(_B, _K, _S, _S)
    mean_ray = jnp.mean(valid_ray, axis=(1, 2))
    mean_pts = jnp.mean((idx_o >= 0).astype(jnp.float32), axis=(1, 2, 3))
    return (feats_out, z_weighted, valid_ray, mean_ray, mean_pts,
            idx_o, zbuf_o, dist_o, w_o)


# speedup vs baseline: 2.9638x; 2.9638x over previous
import functools

import jax
import jax.numpy as jnp
from jax import lax
from jax.experimental import pallas as pl
from jax.experimental.pallas import tpu as pltpu
from jax.experimental.pallas import tpu_sc as plsc

_S = 64
_K = 8
_RADIUS = 2.0
_R = 2.0 * _RADIUS / float(_S)
_R2 = _R * _R
_B = 2
_N = 8192
_F = 64
_P = _S * _S
_NB = 16
_TP = _P // _NB
_CAPL = 168
_LSTRIDE = 176
_CAP = 16 * _LSTRIDE
_GCH = 128
_MARGIN = 0.15625


def _sc_bin_body(xs_hbm, ys_hbm, zs_hbm,
                 bx_hbm, by_hbm, bz_hbm, bi_hbm,
                 xs_v, ys_v, zs_v, bx_v, by_v, bz_v, bi_v):
    b = lax.axis_index("c")
    t = lax.axis_index("s")
    pltpu.sync_copy(xs_hbm.at[b, 0], xs_v)
    pltpu.sync_copy(ys_hbm.at[b, 0], ys_v)
    pltpu.sync_copy(zs_hbm.at[b, 0], zs_v)
    tf = t.astype(jnp.float32)
    hi = (1.0 - (8.0 * tf + 1.0) * 0.015625) + _MARGIN
    lo = (1.0 - (8.0 * tf + 7.0) * 0.015625) - _MARGIN
    base = b * _N

    zeros16 = jnp.zeros((16,), jnp.float32)
    negone16 = jnp.full((16,), -1.0, jnp.float32)
    base16 = jnp.zeros((16,), jnp.int32) + base

    def prefill(j, carry):
        sl = pl.ds(j * 16, 16)
        bx_v[sl] = zeros16
        by_v[sl] = zeros16
        bz_v[sl] = negone16
        bi_v[sl] = base16
        return carry
    lax.fori_loop(0, _CAP // 16, prefill, jnp.int32(0))

    iota16 = lax.iota(jnp.int32, 16)

    lane_base = iota16 * _LSTRIDE

    def scan(i, cntv):
        sl = pl.ds(i * 16, 16)
        y = ys_v[sl]
        m = (y >= lo) & (y <= hi)
        pos = lane_base + jnp.where(m, jnp.minimum(cntv, _CAPL - 1), _CAPL)
        plsc.store_scatter(bx_v, [pos], xs_v[sl])
        plsc.store_scatter(by_v, [pos], y)
        plsc.store_scatter(bz_v, [pos], zs_v[sl])
        plsc.store_scatter(bi_v, [pos], base16 + (i * 16 + iota16))
        return cntv + m.astype(jnp.int32)

    lax.fori_loop(0, _N // 16, scan, jnp.zeros((16,), jnp.int32))

    lane_base = iota16 * _LSTRIDE


    pltpu.sync_copy(bx_v.at[pl.ds(0, _CAP)], bx_hbm.at[b, t, 0])
    pltpu.sync_copy(by_v.at[pl.ds(0, _CAP)], by_hbm.at[b, t, 0])
    pltpu.sync_copy(bz_v.at[pl.ds(0, _CAP)], bz_hbm.at[b, t, 0])
    pltpu.sync_copy(bi_v.at[pl.ds(0, _CAP)], bi_hbm.at[b, t, 0])


def _sc_bin(xs, ys, zs):
    mesh = plsc.VectorSubcoreMesh(core_axis_name="c", subcore_axis_name="s")
    out_type = (
        jax.ShapeDtypeStruct((_B, _NB, 1, _CAP), jnp.float32),
        jax.ShapeDtypeStruct((_B, _NB, 1, _CAP), jnp.float32),
        jax.ShapeDtypeStruct((_B, _NB, 1, _CAP), jnp.float32),
        jax.ShapeDtypeStruct((_B, _NB, 1, _CAP), jnp.int32),
    )
    scratch_types = [
        pltpu.VMEM((_N,), jnp.float32),
        pltpu.VMEM((_N,), jnp.float32),
        pltpu.VMEM((_N,), jnp.float32),
        pltpu.VMEM((_CAP,), jnp.float32),
        pltpu.VMEM((_CAP,), jnp.float32),
        pltpu.VMEM((_CAP,), jnp.float32),
        pltpu.VMEM((_CAP,), jnp.int32),
    ]
    fn = functools.partial(
        pl.kernel, mesh=mesh, out_type=out_type, scratch_types=scratch_types,
        compiler_params=pltpu.CompilerParams(needs_layout_passes=False),
    )(_sc_bin_body)
    return fn(xs, ys, zs)


def _raster_body(bx_ref, by_ref, bz_ref, bi_ref, feats_ref, fout_ref,
                 zw_ref, vray_ref, idx_ref, zbuf_ref, dist_ref, w_ref):
    b = pl.program_id(0)
    t = pl.program_id(1)

    pix_lin = lax.broadcasted_iota(jnp.int32, (_TP, 1), 0)
    row = t * 4 + pix_lin // _S
    col = pix_lin % _S
    pixx = 1.0 - (2.0 * col.astype(jnp.float32) + 1.0) / float(_S)
    pixy = 1.0 - (2.0 * row.astype(jnp.float32) + 1.0) / float(_S)

    ptx = bx_ref[0, 0]
    pty = by_ref[0, 0]
    ptz = bz_ref[0, 0]
    gidx = bi_ref[0, 0]

    a = pixx * pixx + pixy * pixy
    bb = ptx * ptx + pty * pty
    bf = lambda v: v.astype(jnp.bfloat16).astype(jnp.float32)
    c = bf(pixx) * bf(ptx) + bf(pixy) * bf(pty)
    d2 = (a + bb) - 2.0 * c

    valid = (d2 <= _R2) & (ptz > 0.0)
    zbig = jnp.where(valid, jnp.broadcast_to(ptz, d2.shape), jnp.inf)
    big = jnp.int32(1 << 30)

    sel_pairs = []
    idx_cols = []
    zbuf_cols = []
    dist_cols = []
    w_cols = []
    found0 = None
    for _ in range(_K):
        zmin = jnp.min(zbig, axis=1, keepdims=True)
        found = zmin < jnp.inf
        eq = zbig == zmin
        amin = jnp.min(jnp.where(eq, gidx, big), axis=1,
                       keepdims=True)
        onehot = eq & (gidx == amin)
        d2sel = jnp.max(jnp.where(onehot, d2, -1.0), axis=1,
                        keepdims=True)
        zbig = jnp.where(onehot, jnp.inf, zbig)
        wsel = jnp.clip(jnp.exp(-jnp.maximum(d2sel / _R2, 0.0)), 0.0, 0.99)
        sel_pairs.append((jnp.where(found, amin, -1), wsel))
        idx_cols.append(jnp.where(found, amin, -1))
        zbuf_cols.append(jnp.where(found, zmin, -1.0))
        dist_cols.append(jnp.where(found, d2sel, -1.0))
        w_cols.append(jnp.where(found, wsel, 0.99))
        if found0 is None:
            found0 = found

    zbuf_tk = jnp.concatenate(zbuf_cols, axis=1)
    dist_tk = jnp.concatenate(dist_cols, axis=1)
    idx_tk = jnp.concatenate(idx_cols, axis=1)
    w_tk = jnp.concatenate(w_cols, axis=1)

    iota_n = lax.broadcasted_iota(jnp.int32, (_TP, _N), 1) + b * _N
    wfull = jnp.zeros((_TP, _N), jnp.float32)
    for amin, wk in sel_pairs:
        wfull = jnp.where(iota_n == amin, wk, wfull)
    fout = lax.dot_general(feats_ref[0], wfull,
                           dimension_numbers=(((0,), (1,)), ((), ())),
                           preferred_element_type=jnp.float32)
    fout_ref[0] = fout

    wn = jnp.where(idx_tk >= 0, w_tk, 0.0)
    denom = jnp.maximum(jnp.sum(wn, axis=1, keepdims=True), 1e-9)
    wn = wn / denom
    zw = jnp.sum(zbuf_tk * wn, axis=1, keepdims=True)

    zw_ref[0] = zw.reshape(1, _TP)
    vray_ref[0] = found0.astype(jnp.float32).reshape(1, _TP)
    idx_ref[0] = idx_tk.T
    zbuf_ref[0] = zbuf_tk
    dist_ref[0] = dist_tk
    w_ref[0] = w_tk.T


@jax.jit
def kernel(points, features):
    xs = points[:, :, 0].reshape(_B, 1, _N)
    ys = points[:, :, 1].reshape(_B, 1, _N)
    zs = points[:, :, 2].reshape(_B, 1, _N)
    bx, by, bz, bi = _sc_bin(xs, ys, zs)

    grid = (_B, _NB)
    out_shapes = (
        jax.ShapeDtypeStruct((_B, _F, _P), jnp.float32),
        jax.ShapeDtypeStruct((_B, 1, _P), jnp.float32),
        jax.ShapeDtypeStruct((_B, 1, _P), jnp.float32),
        jax.ShapeDtypeStruct((_B, _K, _P), jnp.int32),
        jax.ShapeDtypeStruct((_B, _P, _K), jnp.float32),
        jax.ShapeDtypeStruct((_B, _P, _K), jnp.float32),
        jax.ShapeDtypeStruct((_B, _K, _P), jnp.float32),
    )
    in_specs = [
        pl.BlockSpec((1, 1, 1, _CAP), lambda b, t: (b, t, 0, 0)),
        pl.BlockSpec((1, 1, 1, _CAP), lambda b, t: (b, t, 0, 0)),
        pl.BlockSpec((1, 1, 1, _CAP), lambda b, t: (b, t, 0, 0)),
        pl.BlockSpec((1, 1, 1, _CAP), lambda b, t: (b, t, 0, 0)),
        pl.BlockSpec((1, _N, _F), lambda b, t: (b, 0, 0)),
    ]
    out_specs = (
        pl.BlockSpec((1, _F, _TP), lambda b, t: (b, 0, t)),
        pl.BlockSpec((1, 1, _TP), lambda b, t: (b, 0, t)),
        pl.BlockSpec((1, 1, _TP), lambda b, t: (b, 0, t)),
        pl.BlockSpec((1, _K, _TP), lambda b, t: (b, 0, t)),
        pl.BlockSpec((1, _TP, _K), lambda b, t: (b, t, 0)),
        pl.BlockSpec((1, _TP, _K), lambda b, t: (b, t, 0)),
        pl.BlockSpec((1, _K, _TP), lambda b, t: (b, 0, t)),
    )
    fout, zw, vray, idx, zbuf, dist, w = pl.pallas_call(
        _raster_body,
        grid=grid,
        in_specs=in_specs,
        out_specs=out_specs,
        out_shape=out_shapes,
    )(bx, by, bz, bi, features)

    feats_out = fout.reshape(_B, _F, _S, _S)
    z_weighted = zw.reshape(_B, 1, _S, _S)
    valid_ray = vray.reshape(_B, _S, _S)
    idx_o = idx.reshape(_B, _K, _S, _S)
    zbuf_o = zbuf.reshape(_B, _S, _S, _K)
    dist_o = dist.reshape(_B, _S, _S, _K)
    w_o = w.reshape(_B, _K, _S, _S)
    mean_ray = jnp.mean(valid_ray, axis=(1, 2))
    mean_pts = jnp.mean((idx_o >= 0).astype(jnp.float32), axis=(1, 2, 3))
    return (feats_out, z_weighted, valid_ray, mean_ray, mean_pts,
            idx_o, zbuf_o, dist_o, w_o)
